# baseline (speedup 1.0000x reference)
import functools

import jax
import jax.numpy as jnp
from jax import lax
from jax.experimental import pallas as pl
from jax.experimental.pallas import tpu as pltpu
from jax.experimental.pallas import tpu_sc as plsc

NC = 2
NS = 16
NW = NC * NS
CHUNK = 128


def _ceil_to(v, m):
    return (v + m - 1) // m * m


@functools.lru_cache(maxsize=None)
def _make_agg(n_pad, d, chunks_per_tile):
    rows_per_tile = n_pad // NS
    mesh = plsc.VectorSubcoreMesh(core_axis_name="c", subcore_axis_name="s")

    nc = chunks_per_tile
    nh = nc // 2
    assert nc % 4 == 0 and nc >= 8

    @functools.partial(
        pl.kernel,
        out_type=jax.ShapeDtypeStruct((NC * n_pad, d), jnp.float32),
        mesh=mesh,
        scratch_types=[
            pltpu.VMEM((nh, CHUNK), jnp.int32),
            pltpu.VMEM((nh, CHUNK), jnp.int32),
            pltpu.VMEM((2 * CHUNK, d), jnp.float32),
            pltpu.SemaphoreType.DMA,
            pltpu.SemaphoreType.DMA,
            pltpu.VMEM_SHARED((n_pad, d), jnp.float32),
        ],
        compiler_params=pltpu.CompilerParams(needs_layout_passes=False),
    )
    def agg(table_hbm, src_hbm, dst_hbm, zeros_hbm, out_hbm,
            src_v, dst_v, ring, g0, g1, acc_sh):
        gsem = (g0, g1)
        cid = lax.axis_index("c")
        sid = lax.axis_index("s")
        wid = sid * NC + cid
        pltpu.sync_copy(zeros_hbm,
                        acc_sh.at[pl.ds(sid * rows_per_tile, rows_per_tile)])

        def slot(b):
            return ring.at[pl.ds(b * CHUNK, CHUNK)]

        def gath(j, b):
            pltpu.async_copy(table_hbm.at[src_v.at[j]], slot(b), gsem[b])

        def wait_g(j, b):
            pltpu.make_async_copy(
                table_hbm.at[src_v.at[j]], slot(b), gsem[b]).wait()

        def scat(j, b):
            pltpu.sync_copy(slot(b), acc_sh.at[dst_v.at[j]], add=True)

        for half in range(2):
            base = wid * nc + half * nh
            pltpu.sync_copy(src_hbm.at[pl.ds(base, nh)], src_v)
            pltpu.sync_copy(dst_hbm.at[pl.ds(base, nh)], dst_v)
            if half == 0:
                plsc.subcore_barrier()
            gath(0, 0)

            def body(t, carry):
                for u in range(2):
                    j = t * 2 + u

                    @pl.when(j + 1 < nh)
                    def _(j=j, u=u):
                        gath(j + 1, (u + 1) % 2)
                    wait_g(j, u)
                    scat(j, u)
                return carry

            lax.fori_loop(0, nh // 2, body, 0)
        plsc.subcore_barrier()
        pltpu.sync_copy(
            acc_sh.at[pl.ds(sid * rows_per_tile, rows_per_tile)],
            out_hbm.at[pl.ds(cid * n_pad + sid * rows_per_tile, rows_per_tile)])

    return agg


@functools.lru_cache(maxsize=None)
def _make_decode(d, chunks_per_worker):
    l_per_w = chunks_per_worker * CHUNK
    nj = d // 16
    mesh = plsc.VectorSubcoreMesh(core_axis_name="c", subcore_axis_name="s")

    lc = chunks_per_worker
    assert lc % 2 == 0 and lc >= 4

    @functools.partial(
        pl.kernel,
        out_type=jax.ShapeDtypeStruct((NW * l_per_w,), jnp.float32),
        mesh=mesh,
        scratch_types=[
            pltpu.VMEM((l_per_w,), jnp.int32),
            pltpu.VMEM((l_per_w,), jnp.int32),
            pltpu.VMEM((2 * CHUNK, d), jnp.float32),
            pltpu.VMEM((2 * CHUNK, d), jnp.float32),
            pltpu.VMEM((l_per_w,), jnp.float32),
            pltpu.SemaphoreType.DMA,
            pltpu.SemaphoreType.DMA,
        ],
        compiler_params=pltpu.CompilerParams(needs_layout_passes=False),
    )
    def decode(h_hbm, ia_hbm, ib_hbm, out_hbm, ia_v, ib_v, ra_v, rb_v,
               dots_v, g0, g1):
        gsem = (g0, g1)
        cid = lax.axis_index("c")
        sid = lax.axis_index("s")
        wid = sid * NC + cid
        pltpu.sync_copy(ia_hbm.at[pl.ds(wid * l_per_w, l_per_w)], ia_v)
        pltpu.sync_copy(ib_hbm.at[pl.ds(wid * l_per_w, l_per_w)], ib_v)
        lane = lax.iota(jnp.int32, 16)

        def slots(u):
            return (ra_v.at[pl.ds(u * CHUNK, CHUNK)],
                    rb_v.at[pl.ds(u * CHUNK, CHUNK)])

        def gath(i, u):
            sa, sb = slots(u)
            pltpu.async_copy(
                h_hbm.at[ia_v.at[pl.ds(i * CHUNK, CHUNK)]], sa, gsem[u])
            pltpu.async_copy(
                h_hbm.at[ib_v.at[pl.ds(i * CHUNK, CHUNK)]], sb, gsem[u])

        def wait_g(i, u):
            sa, sb = slots(u)
            pltpu.make_async_copy(
                h_hbm.at[ia_v.at[pl.ds(i * CHUNK, CHUNK)]], sa, gsem[u]).wait()
            pltpu.make_async_copy(
                h_hbm.at[ib_v.at[pl.ds(i * CHUNK, CHUNK)]], sb, gsem[u]).wait()

        def compute(i, u):
            base_r = u * CHUNK

            def group_body(g, c2):
                v = jnp.zeros((16,), jnp.float32)
                for k in range(16):
                    r = base_r + g * 16 + k
                    acc = ra_v[r, pl.ds(0, 16)] * rb_v[r, pl.ds(0, 16)]
                    for j in range(1, nj):
                        acc = acc + (ra_v[r, pl.ds(16 * j, 16)]
                                     * rb_v[r, pl.ds(16 * j, 16)])
                    v = jnp.where(lane == k, jnp.sum(acc), v)
                dots_v[pl.ds(i * CHUNK + g * 16, 16)] = v
                return c2

            lax.fori_loop(0, CHUNK // 16, group_body, 0)

        gath(0, 0)

        def pair_body(t, carry):
            for u in range(2):
                i = t * 2 + u

                @pl.when(i + 1 < lc)
                def _(i=i, u=u):
                    gath(i + 1, (u + 1) % 2)
                wait_g(i, u)
                compute(i, u)
            return carry

        lax.fori_loop(0, lc // 2, pair_body, 0)
        pltpu.sync_copy(dots_v, out_hbm.at[pl.ds(wid * l_per_w, l_per_w)])

    return decode


def _mlp_body(final_relu, x_ref, p0_ref, p1_ref, w1_ref, b1_ref, w2_ref,
              b2_ref, s_ref, t_ref, o_ref):
    a = x_ref[...] + p0_ref[...] + p1_ref[...]
    z = jnp.dot(a, w1_ref[...], preferred_element_type=jnp.float32) + b1_ref[...]
    z = jnp.maximum(z, 0.0)
    z = jnp.dot(z, w2_ref[...], preferred_element_type=jnp.float32) + b2_ref[...]
    z = z * s_ref[...] + t_ref[...]
    if final_relu:
        z = jnp.maximum(z, 0.0)
    o_ref[...] = z


def _mlp(x, p_lo, p_hi, w1, b1, w2, b2, s, t, final_relu, block_rows):
    n, d = x.shape
    d2 = w1.shape[1]
    rb = lambda i: (i, 0)
    full = lambda i: (0, 0)
    return pl.pallas_call(
        functools.partial(_mlp_body, final_relu),
        grid=(n // block_rows,),
        in_specs=[
            pl.BlockSpec((block_rows, d), rb),
            pl.BlockSpec((block_rows, d), rb),
            pl.BlockSpec((block_rows, d), rb),
            pl.BlockSpec((d, d2), full),
            pl.BlockSpec((1, d2), full),
            pl.BlockSpec((d2, d), full),
            pl.BlockSpec((1, d), full),
            pl.BlockSpec((1, d), full),
            pl.BlockSpec((1, d), full),
        ],
        out_specs=pl.BlockSpec((block_rows, d), rb),
        out_shape=jax.ShapeDtypeStruct((n, d), jnp.float32),
    )(x, p_lo, p_hi, w1, b1.reshape(1, d2), w2, b2.reshape(1, d),
      s.reshape(1, d), t.reshape(1, d))


def kernel(x, edge_index, edge_label_index,
           W1_0, b1_0, W2_0, b2_0, bn_g_0, bn_b_0, bn_rm_0, bn_rv_0,
           W1_1, b1_1, W2_1, b2_1, bn_g_1, bn_b_1, bn_rm_1, bn_rv_1):
    n, d = x.shape
    dh = d // 2
    e = edge_index.shape[1]
    l = edge_label_index.shape[1]
    n_pad = _ceil_to(n + 1, NS * 8)
    e_pad = _ceil_to(e, NW * CHUNK * 8)
    l_pad = _ceil_to(l, NW * CHUNK * 2)
    ec = e_pad // (NW * CHUNK)
    lc = l_pad // (NW * CHUNK)

    src = jnp.concatenate(
        [edge_index[0], jnp.zeros((e_pad - e,), jnp.int32)]
    ).reshape(e_pad // CHUNK, CHUNK)
    dst = jnp.concatenate(
        [edge_index[1], jnp.full((e_pad - e,), n, jnp.int32)]
    ).reshape(e_pad // CHUNK, CHUNK)
    zeros_blk = jnp.zeros((n_pad // NS, d), jnp.float32)

    s0 = bn_g_0 * lax.rsqrt(bn_rv_0 + 1e-5)
    t0 = bn_b_0 - bn_rm_0 * s0
    s1 = bn_g_1 * lax.rsqrt(bn_rv_1 + 1e-5)
    t1 = bn_b_1 - bn_rm_1 * s1

    agg = _make_agg(n_pad, d, ec)
    block_rows = 1000 if n % 1000 == 0 else 8
    p = agg(x, src, dst, zeros_blk)
    h0 = _mlp(x, p[:n], p[n_pad:n_pad + n],
              W1_0, b1_0, W2_0, b2_0, s0, t0, True, block_rows)
    p = agg(h0, src, dst, zeros_blk)
    h1 = _mlp(h0, p[:n], p[n_pad:n_pad + n],
              W1_1, b1_1, W2_1, b2_1, s1, t1, False, block_rows)

    ia = jnp.concatenate(
        [edge_label_index[0], jnp.zeros((l_pad - l,), jnp.int32)])
    ib = jnp.concatenate(
        [edge_label_index[1], jnp.zeros((l_pad - l,), jnp.int32)])
    out = _make_decode(d, lc)(h1, ia, ib)
    return out[:l]


# speedup vs baseline: 1.0276x; 1.0276x over previous
import functools

import jax
import jax.numpy as jnp
from jax import lax
from jax.experimental import pallas as pl
from jax.experimental.pallas import tpu as pltpu
from jax.experimental.pallas import tpu_sc as plsc

NC = 2
NS = 16
NW = NC * NS
CHUNK = 128
SPLIT_A = 7
SPLIT_B = 3


def _ceil_to(v, m):
    return (v + m - 1) // m * m


@functools.lru_cache(maxsize=None)
def _make_agg(n_pad, d, wa, wb):
    rows_per_tile = n_pad // NS
    mesh = plsc.VectorSubcoreMesh(core_axis_name="c", subcore_axis_name="s")

    assert wa % 8 == 0 and wb % 8 == 0 and wa >= 4 and wb >= 4
    nh_max = max(wa, wb) // 2

    @functools.partial(
        pl.kernel,
        out_type=jax.ShapeDtypeStruct((NC * n_pad, d), jnp.float32),
        mesh=mesh,
        scratch_types=[
            pltpu.VMEM((nh_max, CHUNK), jnp.int32),
            pltpu.VMEM((nh_max, CHUNK), jnp.int32),
            pltpu.VMEM((2 * CHUNK, d), jnp.float32),
            pltpu.SemaphoreType.DMA,
            pltpu.SemaphoreType.DMA,
            pltpu.VMEM_SHARED((n_pad, d), jnp.float32),
        ],
        compiler_params=pltpu.CompilerParams(needs_layout_passes=False),
    )
    def agg(table_hbm, src_hbm, dst_hbm, zeros_hbm, out_hbm,
            src_v, dst_v, ring, g0, g1, acc_sh):
        gsem = (g0, g1)
        cid = lax.axis_index("c")
        sid = lax.axis_index("s")
        pltpu.sync_copy(zeros_hbm,
                        acc_sh.at[pl.ds(sid * rows_per_tile, rows_per_tile)])
        plsc.subcore_barrier()

        def slot(b):
            return ring.at[pl.ds(b * CHUNK, CHUNK)]

        def gath(j, b):
            pltpu.async_copy(table_hbm.at[src_v.at[j]], slot(b), gsem[b])

        def wait_g(j, b):
            pltpu.make_async_copy(
                table_hbm.at[src_v.at[j]], slot(b), gsem[b]).wait()

        def scat(j, b):
            pltpu.sync_copy(slot(b), acc_sh.at[dst_v.at[j]], add=True)

        def run_core(w, core_base):
            nh = w // 2
            for half in range(2):
                base = core_base + sid * w + half * nh
                pltpu.sync_copy(src_hbm.at[pl.ds(base, nh)],
                                src_v.at[pl.ds(0, nh)])
                pltpu.sync_copy(dst_hbm.at[pl.ds(base, nh)],
                                dst_v.at[pl.ds(0, nh)])
                gath(0, 0)

                def body(t, carry):
                    for u in range(2):
                        j = t * 2 + u

                        @pl.when(j + 1 < nh)
                        def _(j=j, u=u):
                            gath(j + 1, (u + 1) % 2)
                        wait_g(j, u)
                        scat(j, u)
                    return carry

                lax.fori_loop(0, nh // 2, body, 0)

        @pl.when(cid == 0)
        def _():
            run_core(wa, 0)

        @pl.when(cid == 1)
        def _():
            run_core(wb, NS * wa)

        plsc.subcore_barrier()
        pltpu.sync_copy(
            acc_sh.at[pl.ds(sid * rows_per_tile, rows_per_tile)],
            out_hbm.at[pl.ds(cid * n_pad + sid * rows_per_tile, rows_per_tile)])

    return agg


@functools.lru_cache(maxsize=None)
def _make_decode(d, chunks_per_worker):
    l_per_w = chunks_per_worker * CHUNK
    nj = d // 16
    mesh = plsc.VectorSubcoreMesh(core_axis_name="c", subcore_axis_name="s")

    lc = chunks_per_worker
    assert lc % 2 == 0 and lc >= 4

    @functools.partial(
        pl.kernel,
        out_type=jax.ShapeDtypeStruct((NW * l_per_w,), jnp.float32),
        mesh=mesh,
        scratch_types=[
            pltpu.VMEM((l_per_w,), jnp.int32),
            pltpu.VMEM((l_per_w,), jnp.int32),
            pltpu.VMEM((2 * CHUNK, d), jnp.float32),
            pltpu.VMEM((2 * CHUNK, d), jnp.float32),
            pltpu.VMEM((l_per_w,), jnp.float32),
            pltpu.SemaphoreType.DMA,
            pltpu.SemaphoreType.DMA,
        ],
        compiler_params=pltpu.CompilerParams(needs_layout_passes=False),
    )
    def decode(h_hbm, ia_hbm, ib_hbm, out_hbm, ia_v, ib_v, ra_v, rb_v,
               dots_v, g0, g1):
        gsem = (g0, g1)
        cid = lax.axis_index("c")
        sid = lax.axis_index("s")
        wid = sid * NC + cid
        pltpu.sync_copy(ia_hbm.at[pl.ds(wid * l_per_w, l_per_w)], ia_v)
        pltpu.sync_copy(ib_hbm.at[pl.ds(wid * l_per_w, l_per_w)], ib_v)
        lane = lax.iota(jnp.int32, 16)

        def slots(u):
            return (ra_v.at[pl.ds(u * CHUNK, CHUNK)],
                    rb_v.at[pl.ds(u * CHUNK, CHUNK)])

        def gath(i, u):
            sa, sb = slots(u)
            pltpu.async_copy(
                h_hbm.at[ia_v.at[pl.ds(i * CHUNK, CHUNK)]], sa, gsem[u])
            pltpu.async_copy(
                h_hbm.at[ib_v.at[pl.ds(i * CHUNK, CHUNK)]], sb, gsem[u])

        def wait_g(i, u):
            sa, sb = slots(u)
            pltpu.make_async_copy(
                h_hbm.at[ia_v.at[pl.ds(i * CHUNK, CHUNK)]], sa, gsem[u]).wait()
            pltpu.make_async_copy(
                h_hbm.at[ib_v.at[pl.ds(i * CHUNK, CHUNK)]], sb, gsem[u]).wait()

        def compute(i, u):
            base_r = u * CHUNK

            def group_body(g, c2):
                v = jnp.zeros((16,), jnp.float32)
                for k in range(16):
                    r = base_r + g * 16 + k
                    acc = ra_v[r, pl.ds(0, 16)] * rb_v[r, pl.ds(0, 16)]
                    for j in range(1, nj):
                        acc = acc + (ra_v[r, pl.ds(16 * j, 16)]
                                     * rb_v[r, pl.ds(16 * j, 16)])
                    v = jnp.where(lane == k, jnp.sum(acc), v)
                dots_v[pl.ds(i * CHUNK + g * 16, 16)] = v
                return c2

            lax.fori_loop(0, CHUNK // 16, group_body, 0)

        gath(0, 0)

        def pair_body(t, carry):
            for u in range(2):
                i = t * 2 + u

                @pl.when(i + 1 < lc)
                def _(i=i, u=u):
                    gath(i + 1, (u + 1) % 2)
                wait_g(i, u)
                compute(i, u)
            return carry

        lax.fori_loop(0, lc // 2, pair_body, 0)
        pltpu.sync_copy(dots_v, out_hbm.at[pl.ds(wid * l_per_w, l_per_w)])

    return decode


def _mlp_body(final_relu, x_ref, p0_ref, p1_ref, w1_ref, b1_ref, w2_ref,
              b2_ref, s_ref, t_ref, o_ref):
    a = x_ref[...] + p0_ref[...] + p1_ref[...]
    z = jnp.dot(a, w1_ref[...], preferred_element_type=jnp.float32) + b1_ref[...]
    z = jnp.maximum(z, 0.0)
    z = jnp.dot(z, w2_ref[...], preferred_element_type=jnp.float32) + b2_ref[...]
    z = z * s_ref[...] + t_ref[...]
    if final_relu:
        z = jnp.maximum(z, 0.0)
    o_ref[...] = z


def _mlp(x, p_lo, p_hi, w1, b1, w2, b2, s, t, final_relu, block_rows):
    n, d = x.shape
    d2 = w1.shape[1]
    rb = lambda i: (i, 0)
    full = lambda i: (0, 0)
    return pl.pallas_call(
        functools.partial(_mlp_body, final_relu),
        grid=(n // block_rows,),
        in_specs=[
            pl.BlockSpec((block_rows, d), rb),
            pl.BlockSpec((block_rows, d), rb),
            pl.BlockSpec((block_rows, d), rb),
            pl.BlockSpec((d, d2), full),
            pl.BlockSpec((1, d2), full),
            pl.BlockSpec((d2, d), full),
            pl.BlockSpec((1, d), full),
            pl.BlockSpec((1, d), full),
            pl.BlockSpec((1, d), full),
        ],
        out_specs=pl.BlockSpec((block_rows, d), rb),
        out_shape=jax.ShapeDtypeStruct((n, d), jnp.float32),
    )(x, p_lo, p_hi, w1, b1.reshape(1, d2), w2, b2.reshape(1, d),
      s.reshape(1, d), t.reshape(1, d))


def kernel(x, edge_index, edge_label_index,
           W1_0, b1_0, W2_0, b2_0, bn_g_0, bn_b_0, bn_rm_0, bn_rv_0,
           W1_1, b1_1, W2_1, b2_1, bn_g_1, bn_b_1, bn_rm_1, bn_rv_1):
    n, d = x.shape
    dh = d // 2
    e = edge_index.shape[1]
    l = edge_label_index.shape[1]
    n_pad = _ceil_to(n + 1, NS * 8)
    e_pad = _ceil_to(e, NW * CHUNK * 8)
    l_pad = _ceil_to(l, NW * CHUNK * 2)
    ec = e_pad // (NS * CHUNK)
    wa = (ec * SPLIT_A) // (8 * (SPLIT_A + SPLIT_B)) * 8
    wb = ec - wa
    lc = l_pad // (NW * CHUNK)

    src = jnp.concatenate(
        [edge_index[0], jnp.zeros((e_pad - e,), jnp.int32)]
    ).reshape(e_pad // CHUNK, CHUNK)
    dst = jnp.concatenate(
        [edge_index[1], jnp.full((e_pad - e,), n, jnp.int32)]
    ).reshape(e_pad // CHUNK, CHUNK)
    zeros_blk = jnp.zeros((n_pad // NS, d), jnp.float32)

    s0 = bn_g_0 * lax.rsqrt(bn_rv_0 + 1e-5)
    t0 = bn_b_0 - bn_rm_0 * s0
    s1 = bn_g_1 * lax.rsqrt(bn_rv_1 + 1e-5)
    t1 = bn_b_1 - bn_rm_1 * s1

    agg = _make_agg(n_pad, d, wa, wb)
    block_rows = 1000 if n % 1000 == 0 else 8
    p = agg(x, src, dst, zeros_blk)
    h0 = _mlp(x, p[:n], p[n_pad:n_pad + n],
              W1_0, b1_0, W2_0, b2_0, s0, t0, True, block_rows)
    p = agg(h0, src, dst, zeros_blk)
    h1 = _mlp(h0, p[:n], p[n_pad:n_pad + n],
              W1_1, b1_1, W2_1, b2_1, s1, t1, False, block_rows)

    ia = jnp.concatenate(
        [edge_label_index[0], jnp.zeros((l_pad - l,), jnp.int32)])
    ib = jnp.concatenate(
        [edge_label_index[1], jnp.zeros((l_pad - l,), jnp.int32)])
    out = _make_decode(d, lc)(h1, ia, ib)
    return out[:l]


# speedup vs baseline: 1.0318x; 1.0042x over previous
import functools

import jax
import jax.numpy as jnp
from jax import lax
from jax.experimental import pallas as pl
from jax.experimental.pallas import tpu as pltpu
from jax.experimental.pallas import tpu_sc as plsc

NC = 2
NS = 16
NW = NC * NS
CHUNK = 128
SPLIT_A = 4
SPLIT_B = 1


def _ceil_to(v, m):
    return (v + m - 1) // m * m


@functools.lru_cache(maxsize=None)
def _make_agg(n_pad, d, wa, wb):
    rows_per_tile = n_pad // NS
    mesh = plsc.VectorSubcoreMesh(core_axis_name="c", subcore_axis_name="s")

    assert wa % 8 == 0 and wb % 8 == 0 and wa >= 4 and wb >= 4
    nh_max = max(wa, wb) // 2

    @functools.partial(
        pl.kernel,
        out_type=jax.ShapeDtypeStruct((NC * n_pad, d), jnp.float32),
        mesh=mesh,
        scratch_types=[
            pltpu.VMEM((nh_max, CHUNK), jnp.int32),
            pltpu.VMEM((nh_max, CHUNK), jnp.int32),
            pltpu.VMEM((2 * CHUNK, d), jnp.float32),
            pltpu.SemaphoreType.DMA,
            pltpu.SemaphoreType.DMA,
            pltpu.VMEM_SHARED((n_pad, d), jnp.float32),
        ],
        compiler_params=pltpu.CompilerParams(needs_layout_passes=False),
    )
    def agg(table_hbm, src_hbm, dst_hbm, zeros_hbm, out_hbm,
            src_v, dst_v, ring, g0, g1, acc_sh):
        gsem = (g0, g1)
        cid = lax.axis_index("c")
        sid = lax.axis_index("s")
        pltpu.sync_copy(zeros_hbm,
                        acc_sh.at[pl.ds(sid * rows_per_tile, rows_per_tile)])
        plsc.subcore_barrier()

        def slot(b):
            return ring.at[pl.ds(b * CHUNK, CHUNK)]

        def gath(j, b):
            pltpu.async_copy(table_hbm.at[src_v.at[j]], slot(b), gsem[b])

        def wait_g(j, b):
            pltpu.make_async_copy(
                table_hbm.at[src_v.at[j]], slot(b), gsem[b]).wait()

        def scat(j, b):
            pltpu.sync_copy(slot(b), acc_sh.at[dst_v.at[j]], add=True)

        def run_core(w, core_base):
            nh = w // 2
            for half in range(2):
                base = core_base + sid * w + half * nh
                pltpu.sync_copy(src_hbm.at[pl.ds(base, nh)],
                                src_v.at[pl.ds(0, nh)])
                pltpu.sync_copy(dst_hbm.at[pl.ds(base, nh)],
                                dst_v.at[pl.ds(0, nh)])
                gath(0, 0)

                def body(t, carry):
                    for u in range(2):
                        j = t * 2 + u

                        @pl.when(j + 1 < nh)
                        def _(j=j, u=u):
                            gath(j + 1, (u + 1) % 2)
                        wait_g(j, u)
                        scat(j, u)
                    return carry

                lax.fori_loop(0, nh // 2, body, 0)

        @pl.when(cid == 0)
        def _():
            run_core(wa, 0)

        @pl.when(cid == 1)
        def _():
            run_core(wb, NS * wa)

        plsc.subcore_barrier()
        pltpu.sync_copy(
            acc_sh.at[pl.ds(sid * rows_per_tile, rows_per_tile)],
            out_hbm.at[pl.ds(cid * n_pad + sid * rows_per_tile, rows_per_tile)])

    return agg


@functools.lru_cache(maxsize=None)
def _make_decode(d, chunks_per_worker):
    l_per_w = chunks_per_worker * CHUNK
    nj = d // 16
    mesh = plsc.VectorSubcoreMesh(core_axis_name="c", subcore_axis_name="s")

    lc = chunks_per_worker
    assert lc % 2 == 0 and lc >= 4

    @functools.partial(
        pl.kernel,
        out_type=jax.ShapeDtypeStruct((NW * l_per_w,), jnp.float32),
        mesh=mesh,
        scratch_types=[
            pltpu.VMEM((l_per_w,), jnp.int32),
            pltpu.VMEM((l_per_w,), jnp.int32),
            pltpu.VMEM((2 * CHUNK, d), jnp.float32),
            pltpu.VMEM((2 * CHUNK, d), jnp.float32),
            pltpu.VMEM((l_per_w,), jnp.float32),
            pltpu.SemaphoreType.DMA,
            pltpu.SemaphoreType.DMA,
        ],
        compiler_params=pltpu.CompilerParams(needs_layout_passes=False),
    )
    def decode(h_hbm, ia_hbm, ib_hbm, out_hbm, ia_v, ib_v, ra_v, rb_v,
               dots_v, g0, g1):
        gsem = (g0, g1)
        cid = lax.axis_index("c")
        sid = lax.axis_index("s")
        wid = sid * NC + cid
        pltpu.sync_copy(ia_hbm.at[pl.ds(wid * l_per_w, l_per_w)], ia_v)
        pltpu.sync_copy(ib_hbm.at[pl.ds(wid * l_per_w, l_per_w)], ib_v)
        lane = lax.iota(jnp.int32, 16)

        def slots(u):
            return (ra_v.at[pl.ds(u * CHUNK, CHUNK)],
                    rb_v.at[pl.ds(u * CHUNK, CHUNK)])

        def gath(i, u):
            sa, sb = slots(u)
            pltpu.async_copy(
                h_hbm.at[ia_v.at[pl.ds(i * CHUNK, CHUNK)]], sa, gsem[u])
            pltpu.async_copy(
                h_hbm.at[ib_v.at[pl.ds(i * CHUNK, CHUNK)]], sb, gsem[u])

        def wait_g(i, u):
            sa, sb = slots(u)
            pltpu.make_async_copy(
                h_hbm.at[ia_v.at[pl.ds(i * CHUNK, CHUNK)]], sa, gsem[u]).wait()
            pltpu.make_async_copy(
                h_hbm.at[ib_v.at[pl.ds(i * CHUNK, CHUNK)]], sb, gsem[u]).wait()

        def compute(i, u):
            base_r = u * CHUNK

            def group_body(g, c2):
                v = jnp.zeros((16,), jnp.float32)
                for k in range(16):
                    r = base_r + g * 16 + k
                    acc = ra_v[r, pl.ds(0, 16)] * rb_v[r, pl.ds(0, 16)]
                    for j in range(1, nj):
                        acc = acc + (ra_v[r, pl.ds(16 * j, 16)]
                                     * rb_v[r, pl.ds(16 * j, 16)])
                    v = jnp.where(lane == k, jnp.sum(acc), v)
                dots_v[pl.ds(i * CHUNK + g * 16, 16)] = v
                return c2

            lax.fori_loop(0, CHUNK // 16, group_body, 0)

        gath(0, 0)

        def pair_body(t, carry):
            for u in range(2):
                i = t * 2 + u

                @pl.when(i + 1 < lc)
                def _(i=i, u=u):
                    gath(i + 1, (u + 1) % 2)
                wait_g(i, u)
                compute(i, u)
            return carry

        lax.fori_loop(0, lc // 2, pair_body, 0)
        pltpu.sync_copy(dots_v, out_hbm.at[pl.ds(wid * l_per_w, l_per_w)])

    return decode


def _mlp_body(final_relu, x_ref, p0_ref, p1_ref, w1_ref, b1_ref, w2_ref,
              b2_ref, s_ref, t_ref, o_ref):
    a = x_ref[...] + p0_ref[...] + p1_ref[...]
    z = jnp.dot(a, w1_ref[...], preferred_element_type=jnp.float32) + b1_ref[...]
    z = jnp.maximum(z, 0.0)
    z = jnp.dot(z, w2_ref[...], preferred_element_type=jnp.float32) + b2_ref[...]
    z = z * s_ref[...] + t_ref[...]
    if final_relu:
        z = jnp.maximum(z, 0.0)
    o_ref[...] = z


def _mlp(x, p_lo, p_hi, w1, b1, w2, b2, s, t, final_relu, block_rows):
    n, d = x.shape
    d2 = w1.shape[1]
    rb = lambda i: (i, 0)
    full = lambda i: (0, 0)
    return pl.pallas_call(
        functools.partial(_mlp_body, final_relu),
        grid=(n // block_rows,),
        in_specs=[
            pl.BlockSpec((block_rows, d), rb),
            pl.BlockSpec((block_rows, d), rb),
            pl.BlockSpec((block_rows, d), rb),
            pl.BlockSpec((d, d2), full),
            pl.BlockSpec((1, d2), full),
            pl.BlockSpec((d2, d), full),
            pl.BlockSpec((1, d), full),
            pl.BlockSpec((1, d), full),
            pl.BlockSpec((1, d), full),
        ],
        out_specs=pl.BlockSpec((block_rows, d), rb),
        out_shape=jax.ShapeDtypeStruct((n, d), jnp.float32),
    )(x, p_lo, p_hi, w1, b1.reshape(1, d2), w2, b2.reshape(1, d),
      s.reshape(1, d), t.reshape(1, d))


def kernel(x, edge_index, edge_label_index,
           W1_0, b1_0, W2_0, b2_0, bn_g_0, bn_b_0, bn_rm_0, bn_rv_0,
           W1_1, b1_1, W2_1, b2_1, bn_g_1, bn_b_1, bn_rm_1, bn_rv_1):
    n, d = x.shape
    dh = d // 2
    e = edge_index.shape[1]
    l = edge_label_index.shape[1]
    n_pad = _ceil_to(n + 1, NS * 8)
    e_pad = _ceil_to(e, NW * CHUNK * 8)
    l_pad = _ceil_to(l, NW * CHUNK * 2)
    ec = e_pad // (NS * CHUNK)
    wa = (ec * SPLIT_A) // (16 * (SPLIT_A + SPLIT_B)) * 16
    wb = ec - wa
    lc = l_pad // (NW * CHUNK)

    src = jnp.concatenate(
        [edge_index[0], jnp.zeros((e_pad - e,), jnp.int32)]
    ).reshape(e_pad // CHUNK, CHUNK)
    dst = jnp.concatenate(
        [edge_index[1], jnp.full((e_pad - e,), n, jnp.int32)]
    ).reshape(e_pad // CHUNK, CHUNK)
    zeros_blk = jnp.zeros((n_pad // NS, d), jnp.float32)

    s0 = bn_g_0 * lax.rsqrt(bn_rv_0 + 1e-5)
    t0 = bn_b_0 - bn_rm_0 * s0
    s1 = bn_g_1 * lax.rsqrt(bn_rv_1 + 1e-5)
    t1 = bn_b_1 - bn_rm_1 * s1

    agg = _make_agg(n_pad, d, wa, wb)
    block_rows = 1000 if n % 1000 == 0 else 8
    p = agg(x, src, dst, zeros_blk)
    h0 = _mlp(x, p[:n], p[n_pad:n_pad + n],
              W1_0, b1_0, W2_0, b2_0, s0, t0, True, block_rows)
    p = agg(h0, src, dst, zeros_blk)
    h1 = _mlp(h0, p[:n], p[n_pad:n_pad + n],
              W1_1, b1_1, W2_1, b2_1, s1, t1, False, block_rows)

    ia = jnp.concatenate(
        [edge_label_index[0], jnp.zeros((l_pad - l,), jnp.int32)])
    ib = jnp.concatenate(
        [edge_label_index[1], jnp.zeros((l_pad - l,), jnp.int32)])
    out = _make_decode(d, lc)(h1, ia, ib)
    return out[:l]


# speedup vs baseline: 1.0383x; 1.0063x over previous
import functools

import jax
import jax.numpy as jnp
from jax import lax
from jax.experimental import pallas as pl
from jax.experimental.pallas import tpu as pltpu
from jax.experimental.pallas import tpu_sc as plsc

NC = 2
NS = 16
NW = NC * NS
CHUNK = 128
SPLIT_A = 4
SPLIT_B = 1


def _ceil_to(v, m):
    return (v + m - 1) // m * m


@functools.lru_cache(maxsize=None)
def _make_agg(n_pad, d, wa, wb):
    rows_per_tile = n_pad // NS
    mesh = plsc.VectorSubcoreMesh(core_axis_name="c", subcore_axis_name="s")

    assert wa % 8 == 0 and wb % 8 == 0 and wa >= 4 and wb >= 4
    nh_max = max(wa, wb) // 2

    @functools.partial(
        pl.kernel,
        out_type=jax.ShapeDtypeStruct((NC * n_pad, d), jnp.float32),
        mesh=mesh,
        scratch_types=[
            pltpu.VMEM((nh_max, CHUNK), jnp.int32),
            pltpu.VMEM((nh_max, CHUNK), jnp.int32),
            pltpu.VMEM((2 * CHUNK, d), jnp.float32),
            pltpu.SemaphoreType.DMA,
            pltpu.SemaphoreType.DMA,
            pltpu.VMEM_SHARED((n_pad, d), jnp.float32),
        ],
        compiler_params=pltpu.CompilerParams(needs_layout_passes=False),
    )
    def agg(table_hbm, src_hbm, dst_hbm, out_hbm,
            src_v, dst_v, ring, g0, g1, acc_sh):
        gsem = (g0, g1)
        cid = lax.axis_index("c")
        sid = lax.axis_index("s")

        def zrow(r, carry):
            for j in range(d // 16):
                ring[r, pl.ds(16 * j, 16)] = jnp.zeros((16,), jnp.float32)
            return carry

        lax.fori_loop(0, CHUNK, zrow, 0)
        row0 = sid * rows_per_tile
        nfull = rows_per_tile // CHUNK
        for q in range(nfull):
            pltpu.sync_copy(ring.at[pl.ds(0, CHUNK)],
                            acc_sh.at[pl.ds(row0 + q * CHUNK, CHUNK)])
        rem = rows_per_tile - nfull * CHUNK
        if rem:
            pltpu.sync_copy(
                ring.at[pl.ds(0, rem)],
                acc_sh.at[pl.ds(row0 + nfull * CHUNK, rem)])
        plsc.subcore_barrier()

        def slot(b):
            return ring.at[pl.ds(b * CHUNK, CHUNK)]

        def gath(j, b):
            pltpu.async_copy(table_hbm.at[src_v.at[j]], slot(b), gsem[b])

        def wait_g(j, b):
            pltpu.make_async_copy(
                table_hbm.at[src_v.at[j]], slot(b), gsem[b]).wait()

        def scat(j, b):
            pltpu.sync_copy(slot(b), acc_sh.at[dst_v.at[j]], add=True)

        def run_core(w, core_base):
            nh = w // 2
            for half in range(2):
                base = core_base + sid * w + half * nh
                pltpu.sync_copy(src_hbm.at[pl.ds(base, nh)],
                                src_v.at[pl.ds(0, nh)])
                pltpu.sync_copy(dst_hbm.at[pl.ds(base, nh)],
                                dst_v.at[pl.ds(0, nh)])
                gath(0, 0)

                def body(t, carry):
                    for u in range(2):
                        j = t * 2 + u

                        @pl.when(j + 1 < nh)
                        def _(j=j, u=u):
                            gath(j + 1, (u + 1) % 2)
                        wait_g(j, u)
                        scat(j, u)
                    return carry

                lax.fori_loop(0, nh // 2, body, 0)

        @pl.when(cid == 0)
        def _():
            run_core(wa, 0)

        @pl.when(cid == 1)
        def _():
            run_core(wb, NS * wa)

        plsc.subcore_barrier()
        pltpu.sync_copy(
            acc_sh.at[pl.ds(sid * rows_per_tile, rows_per_tile)],
            out_hbm.at[pl.ds(cid * n_pad + sid * rows_per_tile, rows_per_tile)])

    return agg


@functools.lru_cache(maxsize=None)
def _make_decode(d, chunks_per_worker):
    l_per_w = chunks_per_worker * CHUNK
    nj = d // 16
    mesh = plsc.VectorSubcoreMesh(core_axis_name="c", subcore_axis_name="s")

    lc = chunks_per_worker
    assert lc % 2 == 0 and lc >= 4

    @functools.partial(
        pl.kernel,
        out_type=jax.ShapeDtypeStruct((NW * l_per_w,), jnp.float32),
        mesh=mesh,
        scratch_types=[
            pltpu.VMEM((l_per_w,), jnp.int32),
            pltpu.VMEM((l_per_w,), jnp.int32),
            pltpu.VMEM((2 * CHUNK, d), jnp.float32),
            pltpu.VMEM((2 * CHUNK, d), jnp.float32),
            pltpu.VMEM((l_per_w,), jnp.float32),
            pltpu.SemaphoreType.DMA,
            pltpu.SemaphoreType.DMA,
        ],
        compiler_params=pltpu.CompilerParams(needs_layout_passes=False),
    )
    def decode(h_hbm, ia_hbm, ib_hbm, out_hbm, ia_v, ib_v, ra_v, rb_v,
               dots_v, g0, g1):
        gsem = (g0, g1)
        cid = lax.axis_index("c")
        sid = lax.axis_index("s")
        wid = sid * NC + cid
        pltpu.sync_copy(ia_hbm.at[pl.ds(wid * l_per_w, l_per_w)], ia_v)
        pltpu.sync_copy(ib_hbm.at[pl.ds(wid * l_per_w, l_per_w)], ib_v)
        lane = lax.iota(jnp.int32, 16)

        def slots(u):
            return (ra_v.at[pl.ds(u * CHUNK, CHUNK)],
                    rb_v.at[pl.ds(u * CHUNK, CHUNK)])

        def gath(i, u):
            sa, sb = slots(u)
            pltpu.async_copy(
                h_hbm.at[ia_v.at[pl.ds(i * CHUNK, CHUNK)]], sa, gsem[u])
            pltpu.async_copy(
                h_hbm.at[ib_v.at[pl.ds(i * CHUNK, CHUNK)]], sb, gsem[u])

        def wait_g(i, u):
            sa, sb = slots(u)
            pltpu.make_async_copy(
                h_hbm.at[ia_v.at[pl.ds(i * CHUNK, CHUNK)]], sa, gsem[u]).wait()
            pltpu.make_async_copy(
                h_hbm.at[ib_v.at[pl.ds(i * CHUNK, CHUNK)]], sb, gsem[u]).wait()

        def compute(i, u):
            base_r = u * CHUNK

            def group_body(g, c2):
                v = jnp.zeros((16,), jnp.float32)
                for k in range(16):
                    r = base_r + g * 16 + k
                    acc = ra_v[r, pl.ds(0, 16)] * rb_v[r, pl.ds(0, 16)]
                    for j in range(1, nj):
                        acc = acc + (ra_v[r, pl.ds(16 * j, 16)]
                                     * rb_v[r, pl.ds(16 * j, 16)])
                    v = jnp.where(lane == k, jnp.sum(acc), v)
                dots_v[pl.ds(i * CHUNK + g * 16, 16)] = v
                return c2

            lax.fori_loop(0, CHUNK // 16, group_body, 0)

        gath(0, 0)

        def pair_body(t, carry):
            for u in range(2):
                i = t * 2 + u

                @pl.when(i + 1 < lc)
                def _(i=i, u=u):
                    gath(i + 1, (u + 1) % 2)
                wait_g(i, u)
                compute(i, u)
            return carry

        lax.fori_loop(0, lc // 2, pair_body, 0)
        pltpu.sync_copy(dots_v, out_hbm.at[pl.ds(wid * l_per_w, l_per_w)])

    return decode


def _mlp_body(final_relu, x_ref, p0_ref, p1_ref, w1_ref, b1_ref, w2_ref,
              b2_ref, s_ref, t_ref, o_ref):
    a = x_ref[...] + p0_ref[...] + p1_ref[...]
    z = jnp.dot(a, w1_ref[...], preferred_element_type=jnp.float32) + b1_ref[...]
    z = jnp.maximum(z, 0.0)
    z = jnp.dot(z, w2_ref[...], preferred_element_type=jnp.float32) + b2_ref[...]
    z = z * s_ref[...] + t_ref[...]
    if final_relu:
        z = jnp.maximum(z, 0.0)
    o_ref[...] = z


def _mlp(x, p_lo, p_hi, w1, b1, w2, b2, s, t, final_relu, block_rows):
    n, d = x.shape
    d2 = w1.shape[1]
    rb = lambda i: (i, 0)
    full = lambda i: (0, 0)
    return pl.pallas_call(
        functools.partial(_mlp_body, final_relu),
        grid=(n // block_rows,),
        in_specs=[
            pl.BlockSpec((block_rows, d), rb),
            pl.BlockSpec((block_rows, d), rb),
            pl.BlockSpec((block_rows, d), rb),
            pl.BlockSpec((d, d2), full),
            pl.BlockSpec((1, d2), full),
            pl.BlockSpec((d2, d), full),
            pl.BlockSpec((1, d), full),
            pl.BlockSpec((1, d), full),
            pl.BlockSpec((1, d), full),
        ],
        out_specs=pl.BlockSpec((block_rows, d), rb),
        out_shape=jax.ShapeDtypeStruct((n, d), jnp.float32),
    )(x, p_lo, p_hi, w1, b1.reshape(1, d2), w2, b2.reshape(1, d),
      s.reshape(1, d), t.reshape(1, d))


def kernel(x, edge_index, edge_label_index,
           W1_0, b1_0, W2_0, b2_0, bn_g_0, bn_b_0, bn_rm_0, bn_rv_0,
           W1_1, b1_1, W2_1, b2_1, bn_g_1, bn_b_1, bn_rm_1, bn_rv_1):
    n, d = x.shape
    dh = d // 2
    e = edge_index.shape[1]
    l = edge_label_index.shape[1]
    n_pad = _ceil_to(n + 1, NS * 8)
    e_pad = _ceil_to(e, NW * CHUNK * 8)
    l_pad = _ceil_to(l, NW * CHUNK * 2)
    ec = e_pad // (NS * CHUNK)
    wa = (ec * SPLIT_A) // (16 * (SPLIT_A + SPLIT_B)) * 16
    wb = ec - wa
    lc = l_pad // (NW * CHUNK)

    src = jnp.concatenate(
        [edge_index[0], jnp.zeros((e_pad - e,), jnp.int32)]
    ).reshape(e_pad // CHUNK, CHUNK)
    dst = jnp.concatenate(
        [edge_index[1], jnp.full((e_pad - e,), n, jnp.int32)]
    ).reshape(e_pad // CHUNK, CHUNK)

    s0 = bn_g_0 * lax.rsqrt(bn_rv_0 + 1e-5)
    t0 = bn_b_0 - bn_rm_0 * s0
    s1 = bn_g_1 * lax.rsqrt(bn_rv_1 + 1e-5)
    t1 = bn_b_1 - bn_rm_1 * s1

    agg = _make_agg(n_pad, d, wa, wb)
    block_rows = 1000 if n % 1000 == 0 else 8
    p = agg(x, src, dst)
    h0 = _mlp(x, p[:n], p[n_pad:n_pad + n],
              W1_0, b1_0, W2_0, b2_0, s0, t0, True, block_rows)
    p = agg(h0, src, dst)
    h1 = _mlp(h0, p[:n], p[n_pad:n_pad + n],
              W1_1, b1_1, W2_1, b2_1, s1, t1, False, block_rows)

    ia = jnp.concatenate(
        [edge_label_index[0], jnp.zeros((l_pad - l,), jnp.int32)])
    ib = jnp.concatenate(
        [edge_label_index[1], jnp.zeros((l_pad - l,), jnp.int32)])
    out = _make_decode(d, lc)(h1, ia, ib)
    return out[:l]
